# baseline (speedup 1.0000x reference)
import functools

import jax
import jax.numpy as jnp
from jax import lax
from jax.experimental import pallas as pl
from jax.experimental.pallas import tpu as pltpu
from jax.experimental.pallas import tpu_sc as plsc

VOCAB = 1000000
DIM = 64
BATCH = 16384
HIST = 200

NC = 2
NS = 16
NW = NC * NS

RPW = BATCH // NW
CROWS = 4
NCHUNK = RPW // CROWS
NBUF = 2
CW = CROWS * DIM

_mesh = plsc.VectorSubcoreMesh(core_axis_name="c", subcore_axis_name="s")


@functools.partial(
    pl.kernel,
    out_type=jax.ShapeDtypeStruct((HIST, BATCH * DIM), jnp.float32),
    mesh=_mesh,
    scratch_types=[
        pltpu.VMEM((NBUF, CROWS, HIST), jnp.int32),
        pltpu.VMEM((NBUF, CROWS, HIST, DIM), jnp.float32),
        pltpu.SemaphoreType.DMA((NBUF,)),
        pltpu.SemaphoreType.DMA((NBUF,)),
        pltpu.SemaphoreType.DMA((NBUF,)),
    ],
    compiler_params=pltpu.CompilerParams(use_tc_tiling_on_sc=False),
)
def _embed(x_hbm, w_hbm, out_hbm, idx_v, rows_v, isem, gsem, ssem):
    wid = lax.axis_index("s") * NC + lax.axis_index("c")
    row0 = wid * RPW

    def idx_copy(b, j, sem_op):
        sem_op(x_hbm.at[pl.ds(row0 + j * CROWS, CROWS)], idx_v.at[b],
               isem.at[b])

    def fire_gathers(b):
        for k in range(CROWS):
            pltpu.async_copy(w_hbm.at[idx_v.at[b, k]], rows_v.at[b, k],
                             gsem.at[b])

    def wait_gathers(b):
        for k in range(CROWS):
            pltpu.make_async_copy(w_hbm.at[idx_v.at[b, k]], rows_v.at[b, k],
                                  gsem.at[b]).wait()

    def store(b, j, sem_op):
        for k in range(CROWS):
            sem_op(rows_v.at[b, k],
                   out_hbm.at[:, pl.ds((row0 + j * CROWS + k) * DIM, DIM)],
                   ssem.at[b])

    _start = pltpu.async_copy

    def _wait(s, d, m):
        pltpu.make_async_copy(s, d, m).wait()

    for b in range(NBUF):
        idx_copy(b, b, _start)
    for b in range(NBUF):
        idx_copy(b, b, _wait)
        fire_gathers(b)

    def outer(i, carry):
        for b in range(NBUF):
            j = i * NBUF + b
            wait_gathers(b)
            store(b, j, _start)
            idx_copy(b, j + NBUF, _start)
            store(b, j, _wait)
            idx_copy(b, j + NBUF, _wait)
            fire_gathers(b)
        return carry

    lax.fori_loop(0, NCHUNK // NBUF - 1, outer, 0)

    for b in range(NBUF):
        wait_gathers(b)
        store(b, NCHUNK - NBUF + b, _start)
    for b in range(NBUF):
        store(b, NCHUNK - NBUF + b, _wait)


def kernel(x, weight):
    out_t = _embed(x.astype(jnp.int32), weight)
    return jnp.swapaxes(out_t.reshape(HIST, BATCH, DIM), 0, 1)


# speedup vs baseline: 6.5091x; 6.5091x over previous
import functools

import jax
import jax.numpy as jnp
from jax import lax
from jax.experimental import pallas as pl
from jax.experimental.pallas import tpu as pltpu
from jax.experimental.pallas import tpu_sc as plsc

VOCAB = 1000000
DIM = 64
BATCH = 16384
HIST = 200

NC = 2
NS = 16
NW = NC * NS

RPW = BATCH // NW
CROWS = 4
NCHUNK = RPW // CROWS
NBUF = 2

_mesh = plsc.VectorSubcoreMesh(core_axis_name="c", subcore_axis_name="s")


@functools.partial(
    pl.kernel,
    out_type=jax.ShapeDtypeStruct((BATCH, HIST, DIM), jnp.float32),
    mesh=_mesh,
    scratch_types=[
        pltpu.VMEM((NBUF, CROWS, HIST), jnp.int32),
        pltpu.VMEM((NBUF, CROWS, HIST, DIM), jnp.float32),
        pltpu.SemaphoreType.DMA((NBUF,)),
        pltpu.SemaphoreType.DMA((NBUF,)),
        pltpu.SemaphoreType.DMA((NBUF,)),
    ],
    compiler_params=pltpu.CompilerParams(use_tc_tiling_on_sc=False),
)
def _embed(x_hbm, w_hbm, out_hbm, idx_v, rows_v, isem, gsem, ssem):
    wid = lax.axis_index("s") * NC + lax.axis_index("c")
    row0 = wid * RPW

    def idx_copy(b, j, sem_op):
        sem_op(x_hbm.at[pl.ds(row0 + j * CROWS, CROWS)], idx_v.at[b],
               isem.at[b])

    def fire_gathers(b):
        for k in range(CROWS):
            pltpu.async_copy(w_hbm.at[idx_v.at[b, k]], rows_v.at[b, k],
                             gsem.at[b])

    def wait_gathers(b):
        for k in range(CROWS):
            pltpu.make_async_copy(w_hbm.at[idx_v.at[b, k]], rows_v.at[b, k],
                                  gsem.at[b]).wait()

    def store(b, j, sem_op):
        sem_op(rows_v.at[b], out_hbm.at[pl.ds(row0 + j * CROWS, CROWS)],
               ssem.at[b])

    _start = pltpu.async_copy

    def _wait(s, d, m):
        pltpu.make_async_copy(s, d, m).wait()

    for b in range(NBUF):
        idx_copy(b, b, _start)
    for b in range(NBUF):
        idx_copy(b, b, _wait)
        fire_gathers(b)

    def outer(i, carry):
        for b in range(NBUF):
            j = i * NBUF + b
            wait_gathers(b)
            store(b, j, _start)
            idx_copy(b, j + NBUF, _start)
            store(b, j, _wait)
            idx_copy(b, j + NBUF, _wait)
            fire_gathers(b)
        return carry

    lax.fori_loop(0, NCHUNK // NBUF - 1, outer, 0)

    for b in range(NBUF):
        wait_gathers(b)
        store(b, NCHUNK - NBUF + b, _start)
    for b in range(NBUF):
        store(b, NCHUNK - NBUF + b, _wait)


def kernel(x, weight):
    return _embed(x.astype(jnp.int32), weight)
